# Initial kernel scaffold; baseline (speedup 1.0000x reference)
import jax
import jax.numpy as jnp
from jax.experimental import pallas as pl


def kernel(x, edge_index, prefix_sum, W1, b1, W2, b2, lin1_W, lin1_b, out_W, out_b):
    raise NotImplementedError("write your pallas kernel here")


# speedup vs baseline: 25.1934x; 25.1934x over previous
import functools

import jax
import jax.numpy as jnp
from jax import lax
from jax.experimental import pallas as pl
from jax.experimental.pallas import tpu as pltpu
from jax.experimental.pallas import tpu_sc as plsc

NS = 16
LANES = 16


@functools.cache
def _sc_edge_kernel(n, e):
    ept = e // NS
    rows = ept // 128
    npt = n // NS
    ncv = npt // LANES

    mesh = plsc.VectorSubcoreMesh(
        core_axis_name="c", subcore_axis_name="s", num_cores=1)

    def body(y_hbm, src_hbm, dst_hbm, b1_hbm, s_hbm, h1_hbm,
             yfull, srcbuf, dstbuf, valbuf, onesbuf,
             abuf, dbuf, hbuf, zbuf, b1buf,
             sh_acc1, sh_deg, sh_acc2):
        wid = lax.axis_index("s")

        pltpu.sync_copy(src_hbm.at[pl.ds(wid * rows, rows)], srcbuf)
        pltpu.sync_copy(dst_hbm.at[pl.ds(wid * rows, rows)], dstbuf)
        pltpu.sync_copy(y_hbm, yfull)
        pltpu.sync_copy(b1_hbm, b1buf)

        for k in range(8):
            onesbuf[pl.ds(k * LANES, LANES)] = jnp.full(
                (LANES,), 1.0, jnp.float32)

        def fill_zero(i, carry):
            zbuf[pl.ds(i * LANES, LANES)] = jnp.zeros((LANES,), jnp.float32)
            return carry
        lax.fori_loop(0, ncv, fill_zero, 0)

        pltpu.sync_copy(zbuf, sh_acc1.at[pl.ds(wid * npt, npt)])
        pltpu.sync_copy(zbuf, sh_deg.at[pl.ds(wid * npt, npt)])
        pltpu.sync_copy(zbuf, sh_acc2.at[pl.ds(wid * npt, npt)])
        plsc.subcore_barrier()

        def gather_chunk(j, carry):
            for k in range(8):
                sv = srcbuf[j, pl.ds(k * LANES, LANES)]
                valbuf[j, pl.ds(k * LANES, LANES)] = plsc.load_gather(
                    yfull, [sv])
            return carry

        def l1_row(j, carry):
            gather_chunk(j, carry)
            pltpu.sync_copy(valbuf.at[j], sh_acc1.at[dstbuf.at[j]], add=True)
            pltpu.sync_copy(onesbuf, sh_deg.at[dstbuf.at[j]], add=True)
            return carry
        lax.fori_loop(0, rows, l1_row, 0)
        plsc.subcore_barrier()

        pltpu.sync_copy(sh_acc1.at[pl.ds(wid * npt, npt)], abuf)
        pltpu.sync_copy(sh_deg.at[pl.ds(wid * npt, npt)], dbuf)
        b1v = b1buf[...]

        def h1_chunk(i, carry):
            a = abuf[pl.ds(i * LANES, LANES)]
            d = dbuf[pl.ds(i * LANES, LANES)]
            hbuf[pl.ds(i * LANES, LANES)] = jnp.maximum(
                a / jnp.maximum(d, 1.0) + b1v, 0.0)
            return carry
        lax.fori_loop(0, ncv, h1_chunk, 0)
        pltpu.sync_copy(hbuf, h1_hbm.at[pl.ds(wid * npt, npt)])
        plsc.subcore_barrier()

        pltpu.sync_copy(h1_hbm, yfull)

        def l2_row(j, carry):
            gather_chunk(j, carry)
            pltpu.sync_copy(valbuf.at[j], sh_acc2.at[dstbuf.at[j]], add=True)
            return carry
        lax.fori_loop(0, rows, l2_row, 0)
        plsc.subcore_barrier()

        pltpu.sync_copy(sh_acc2.at[pl.ds(wid * npt, npt)], abuf)

        def s_chunk(i, carry):
            a = abuf[pl.ds(i * LANES, LANES)]
            d = dbuf[pl.ds(i * LANES, LANES)]
            hbuf[pl.ds(i * LANES, LANES)] = a / jnp.maximum(d, 1.0)
            return carry
        lax.fori_loop(0, ncv, s_chunk, 0)
        pltpu.sync_copy(hbuf, s_hbm.at[pl.ds(wid * npt, npt)])

    return pl.kernel(
        body,
        out_type=[jax.ShapeDtypeStruct((n,), jnp.float32),
                  jax.ShapeDtypeStruct((n,), jnp.float32)],
        mesh=mesh,
        compiler_params=pltpu.CompilerParams(needs_layout_passes=False),
        scratch_types=[
            pltpu.VMEM((n,), jnp.float32),
            pltpu.VMEM((rows, 128), jnp.int32),
            pltpu.VMEM((rows, 128), jnp.int32),
            pltpu.VMEM((rows, 128), jnp.float32),
            pltpu.VMEM((128,), jnp.float32),
            pltpu.VMEM((npt,), jnp.float32),
            pltpu.VMEM((npt,), jnp.float32),
            pltpu.VMEM((npt,), jnp.float32),
            pltpu.VMEM((npt,), jnp.float32),
            pltpu.VMEM((LANES,), jnp.float32),
            pltpu.VMEM_SHARED((n,), jnp.float32),
            pltpu.VMEM_SHARED((n,), jnp.float32),
            pltpu.VMEM_SHARED((n,), jnp.float32),
        ],
    )


def _tc_y(x, w1row):
    n = x.shape[0]
    blk = 2048

    def body(x_ref, w_ref, o_ref):
        o_ref[...] = jnp.sum(x_ref[...] * w_ref[...], axis=1, keepdims=True)

    return pl.pallas_call(
        body,
        grid=(n // blk,),
        in_specs=[pl.BlockSpec((blk, x.shape[1]), lambda i: (i, 0)),
                  pl.BlockSpec((1, x.shape[1]), lambda i: (0, 0))],
        out_specs=pl.BlockSpec((blk, 1), lambda i: (i, 0)),
        out_shape=jax.ShapeDtypeStruct((n, 1), jnp.float32),
    )(x, w1row)


def _tc_final(s3d, u, v, c, ow, ob):
    b, _, n_per = s3d.shape
    hd = u.shape[0]

    def body(s_ref, u_ref, v_ref, c_ref, w_ref, b_ref, o_ref):
        sv = s_ref[...].reshape(1, n_per)
        m = jnp.sum(sv) * (1.0 / n_per)
        base = m * v_ref[...] + c_ref[...]
        h = jnp.maximum(u_ref[...] * sv + base, 0.0)
        o = jnp.sum(w_ref[...] * h, axis=0, keepdims=True) + b_ref[...]
        o_ref[...] = o.reshape(1, 1, n_per)

    return pl.pallas_call(
        body,
        grid=(b,),
        in_specs=[pl.BlockSpec((1, 1, n_per), lambda i: (i, 0, 0)),
                  pl.BlockSpec((hd, 1), lambda i: (0, 0)),
                  pl.BlockSpec((hd, 1), lambda i: (0, 0)),
                  pl.BlockSpec((hd, 1), lambda i: (0, 0)),
                  pl.BlockSpec((hd, 1), lambda i: (0, 0)),
                  pl.BlockSpec((1, 1), lambda i: (0, 0))],
        out_specs=pl.BlockSpec((1, 1, n_per), lambda i: (i, 0, 0)),
        out_shape=jax.ShapeDtypeStruct((b, 1, n_per), jnp.float32),
    )(s3d, u, v, c, ow, ob)


def kernel(x, edge_index, prefix_sum, W1, b1, W2, b2, lin1_W, lin1_b, out_W, out_b):
    n = x.shape[0]
    e = edge_index.shape[1]
    nb = prefix_sum.shape[0]
    n_per = n // nb

    y = _tc_y(x, W1.reshape(1, -1)).reshape(n)
    src2d = edge_index[0].reshape(e // 128, 128)
    dst2d = edge_index[1].reshape(e // 128, 128)
    b1s = jnp.full((LANES,), b1[0], jnp.float32)

    s_flat, _h1 = _sc_edge_kernel(n, e)(y, src2d, dst2d, b1s)

    latent = W2.shape[1]
    w2 = W2[0]
    u = w2 @ lin1_W[:latent, :]
    v = w2 @ lin1_W[latent:, :]
    c = b2 @ (lin1_W[:latent, :] + lin1_W[latent:, :]) + lin1_b

    hd = u.shape[0]
    pred2d = _tc_final(s_flat.reshape(nb, 1, n_per),
                       u.reshape(hd, 1), v.reshape(hd, 1), c.reshape(hd, 1),
                       out_W.reshape(hd, 1), out_b.reshape(1, 1))
    return pred2d.reshape(n, 1)


# speedup vs baseline: 28.4721x; 1.1301x over previous
import functools

import jax
import jax.numpy as jnp
from jax import lax
from jax.experimental import pallas as pl
from jax.experimental.pallas import tpu as pltpu
from jax.experimental.pallas import tpu_sc as plsc

NS = 16
LANES = 16


@functools.cache
def _sc_edge_kernel(n, e):
    ept = e // NS
    rows = ept // 128
    npt = n // NS
    ncv = npt // LANES

    mesh = plsc.VectorSubcoreMesh(
        core_axis_name="c", subcore_axis_name="s", num_cores=1)

    def body(y_hbm, src_hbm, dst_hbm, b1_hbm, s_hbm, h1_hbm,
             yfull, srcbuf, dstbuf, valbuf, onesbuf,
             abuf, dbuf, hbuf, zbuf, b1buf, sem,
             sh_acc1, sh_deg, sh_acc2):
        wid = lax.axis_index("s")

        pltpu.sync_copy(src_hbm.at[pl.ds(wid * rows, rows)], srcbuf)
        pltpu.sync_copy(dst_hbm.at[pl.ds(wid * rows, rows)], dstbuf)
        pltpu.sync_copy(y_hbm, yfull)
        pltpu.sync_copy(b1_hbm, b1buf)

        for k in range(8):
            onesbuf[pl.ds(k * LANES, LANES)] = jnp.full(
                (LANES,), 1.0, jnp.float32)

        def fill_zero(i, carry):
            zbuf[pl.ds(i * LANES, LANES)] = jnp.zeros((LANES,), jnp.float32)
            return carry
        lax.fori_loop(0, ncv, fill_zero, 0)

        pltpu.sync_copy(zbuf, sh_acc1.at[pl.ds(wid * npt, npt)])
        pltpu.sync_copy(zbuf, sh_deg.at[pl.ds(wid * npt, npt)])
        pltpu.sync_copy(zbuf, sh_acc2.at[pl.ds(wid * npt, npt)])
        plsc.subcore_barrier()

        def gather_chunk(j, carry):
            for k in range(8):
                sv = srcbuf[j, pl.ds(k * LANES, LANES)]
                valbuf[j, pl.ds(k * LANES, LANES)] = plsc.load_gather(
                    yfull, [sv])
            return carry

        lax.fori_loop(0, rows, gather_chunk, 0)
        descs = []
        for j in range(rows):
            descs.append(pltpu.async_copy(
                valbuf.at[j], sh_acc1.at[dstbuf.at[j]], sem, add=True))
            descs.append(pltpu.async_copy(
                onesbuf, sh_deg.at[dstbuf.at[j]], sem, add=True))
        for d in descs:
            d.wait()
        plsc.subcore_barrier()

        pltpu.sync_copy(sh_acc1.at[pl.ds(wid * npt, npt)], abuf)
        pltpu.sync_copy(sh_deg.at[pl.ds(wid * npt, npt)], dbuf)
        b1v = b1buf[...]

        def h1_chunk(i, carry):
            a = abuf[pl.ds(i * LANES, LANES)]
            d = dbuf[pl.ds(i * LANES, LANES)]
            hbuf[pl.ds(i * LANES, LANES)] = jnp.maximum(
                a / jnp.maximum(d, 1.0) + b1v, 0.0)
            return carry
        lax.fori_loop(0, ncv, h1_chunk, 0)
        pltpu.sync_copy(hbuf, h1_hbm.at[pl.ds(wid * npt, npt)])
        plsc.subcore_barrier()

        pltpu.sync_copy(h1_hbm, yfull)
        lax.fori_loop(0, rows, gather_chunk, 0)
        descs = []
        for j in range(rows):
            descs.append(pltpu.async_copy(
                valbuf.at[j], sh_acc2.at[dstbuf.at[j]], sem, add=True))
        for d in descs:
            d.wait()
        plsc.subcore_barrier()

        pltpu.sync_copy(sh_acc2.at[pl.ds(wid * npt, npt)], abuf)

        def s_chunk(i, carry):
            a = abuf[pl.ds(i * LANES, LANES)]
            d = dbuf[pl.ds(i * LANES, LANES)]
            hbuf[pl.ds(i * LANES, LANES)] = a / jnp.maximum(d, 1.0)
            return carry
        lax.fori_loop(0, ncv, s_chunk, 0)
        pltpu.sync_copy(hbuf, s_hbm.at[pl.ds(wid * npt, npt)])

    return pl.kernel(
        body,
        out_type=[jax.ShapeDtypeStruct((n,), jnp.float32),
                  jax.ShapeDtypeStruct((n,), jnp.float32)],
        mesh=mesh,
        compiler_params=pltpu.CompilerParams(needs_layout_passes=False),
        scratch_types=[
            pltpu.VMEM((n,), jnp.float32),
            pltpu.VMEM((rows, 128), jnp.int32),
            pltpu.VMEM((rows, 128), jnp.int32),
            pltpu.VMEM((rows, 128), jnp.float32),
            pltpu.VMEM((128,), jnp.float32),
            pltpu.VMEM((npt,), jnp.float32),
            pltpu.VMEM((npt,), jnp.float32),
            pltpu.VMEM((npt,), jnp.float32),
            pltpu.VMEM((npt,), jnp.float32),
            pltpu.VMEM((LANES,), jnp.float32),
            pltpu.SemaphoreType.DMA,
            pltpu.VMEM_SHARED((n,), jnp.float32),
            pltpu.VMEM_SHARED((n,), jnp.float32),
            pltpu.VMEM_SHARED((n,), jnp.float32),
        ],
    )


def _tc_y(x, w1row):
    n = x.shape[0]
    blk = 2048

    def body(x_ref, w_ref, o_ref):
        o_ref[...] = jnp.sum(x_ref[...] * w_ref[...], axis=1, keepdims=True)

    return pl.pallas_call(
        body,
        grid=(n // blk,),
        in_specs=[pl.BlockSpec((blk, x.shape[1]), lambda i: (i, 0)),
                  pl.BlockSpec((1, x.shape[1]), lambda i: (0, 0))],
        out_specs=pl.BlockSpec((blk, 1), lambda i: (i, 0)),
        out_shape=jax.ShapeDtypeStruct((n, 1), jnp.float32),
    )(x, w1row)


def _tc_final(s3d, u, v, c, ow, ob):
    b, _, n_per = s3d.shape
    hd = u.shape[0]

    def body(s_ref, u_ref, v_ref, c_ref, w_ref, b_ref, o_ref):
        sv = s_ref[...].reshape(1, n_per)
        m = jnp.sum(sv) * (1.0 / n_per)
        base = m * v_ref[...] + c_ref[...]
        h = jnp.maximum(u_ref[...] * sv + base, 0.0)
        o = jnp.sum(w_ref[...] * h, axis=0, keepdims=True) + b_ref[...]
        o_ref[...] = o.reshape(1, 1, n_per)

    return pl.pallas_call(
        body,
        grid=(b,),
        in_specs=[pl.BlockSpec((1, 1, n_per), lambda i: (i, 0, 0)),
                  pl.BlockSpec((hd, 1), lambda i: (0, 0)),
                  pl.BlockSpec((hd, 1), lambda i: (0, 0)),
                  pl.BlockSpec((hd, 1), lambda i: (0, 0)),
                  pl.BlockSpec((hd, 1), lambda i: (0, 0)),
                  pl.BlockSpec((1, 1), lambda i: (0, 0))],
        out_specs=pl.BlockSpec((1, 1, n_per), lambda i: (i, 0, 0)),
        out_shape=jax.ShapeDtypeStruct((b, 1, n_per), jnp.float32),
    )(s3d, u, v, c, ow, ob)


def kernel(x, edge_index, prefix_sum, W1, b1, W2, b2, lin1_W, lin1_b, out_W, out_b):
    n = x.shape[0]
    e = edge_index.shape[1]
    nb = prefix_sum.shape[0]
    n_per = n // nb

    y = _tc_y(x, W1.reshape(1, -1)).reshape(n)
    src2d = edge_index[0].reshape(e // 128, 128)
    dst2d = edge_index[1].reshape(e // 128, 128)
    b1s = jnp.full((LANES,), b1[0], jnp.float32)

    s_flat, _h1 = _sc_edge_kernel(n, e)(y, src2d, dst2d, b1s)

    latent = W2.shape[1]
    w2 = W2[0]
    u = w2 @ lin1_W[:latent, :]
    v = w2 @ lin1_W[latent:, :]
    c = b2 @ (lin1_W[:latent, :] + lin1_W[latent:, :]) + lin1_b

    hd = u.shape[0]
    pred2d = _tc_final(s_flat.reshape(nb, 1, n_per),
                       u.reshape(hd, 1), v.reshape(hd, 1), c.reshape(hd, 1),
                       out_W.reshape(hd, 1), out_b.reshape(1, 1))
    return pred2d.reshape(n, 1)
